# baseline (speedup 1.0000x reference)
import functools

import jax
import jax.numpy as jnp
from jax import lax
from jax.experimental import pallas as pl
from jax.experimental.pallas import tpu as pltpu
from jax.experimental.pallas import tpu_sc as plsc

N = 320000
D = 128
S = 1024
CHUNK = 256
SUB = 128
SPAN = 32
HALF = 16
NCHUNKS = N // CHUNK
NC = 2
NS = 16
CHUNKS_PER_SC = NCHUNKS // NC
BASE_PER_TILE = CHUNKS_PER_SC // NS
EXTRA = CHUNKS_PER_SC - BASE_PER_TILE * NS
SEGS_PER_TILE = S // NS


def _accumulate():
    mesh = plsc.VectorSubcoreMesh(core_axis_name="c", subcore_axis_name="s")

    @functools.partial(
        pl.kernel,
        mesh=mesh,
        out_type=[
            jax.ShapeDtypeStruct((NC, S, D), jnp.float32),
            jax.ShapeDtypeStruct((NC, NS, S), jnp.float32),
        ],
        scratch_types=[
            pltpu.VMEM((CHUNK // SUB, SUB), jnp.int32),
            pltpu.VMEM((CHUNK // SUB, SUB), jnp.int32),
            pltpu.VMEM((CHUNK, D), jnp.float32),
            pltpu.VMEM((CHUNK, D), jnp.float32),
            pltpu.VMEM((S + 16,), jnp.float32),
            pltpu.VMEM((HALF, D), jnp.float32),
            pltpu.VMEM((HALF,), jnp.int32),
            pltpu.VMEM((D,), jnp.float32),
            pltpu.VMEM_SHARED((S, D), jnp.float32),
            pltpu.SemaphoreType.DMA,
            pltpu.SemaphoreType.DMA,
            pltpu.SemaphoreType.DMA,
            pltpu.SemaphoreType.DMA,
        ],
    )
    def body(ins_hbm, ids_hbm, sums_out, cnt_out,
             idx_v0, idx_v1, rows_v0, rows_v1, cnt_v, flush_v, fidx_v,
             run_v, acc_sh, rsem0, rsem1, isem0, isem1):
        c = lax.axis_index("c")
        s = lax.axis_index("s")
        zero16 = jnp.zeros((16,), jnp.float32)
        ilane = lax.iota(jnp.int32, 16)
        inc1 = jnp.where(ilane == 0, 1.0, 0.0).astype(jnp.float32)
        idx_bufs = (idx_v0, idx_v1)
        rows_bufs = (rows_v0, rows_v1)
        rsems = (rsem0, rsem1)
        isems = (isem0, isem1)

        def z_rows(i, _):
            rows_v0[i // 8, pl.ds((i % 8) * 16, 16)] = zero16
            return 0
        lax.fori_loop(0, SEGS_PER_TILE * (D // 16), z_rows, 0)

        def z_cnt(i, _):
            cnt_v[pl.ds(i * 16, 16)] = zero16
            return 0
        lax.fori_loop(0, (S + 16) // 16, z_cnt, 0)

        def z_flush(i, _):
            flush_v[i // 8, pl.ds((i % 8) * 16, 16)] = zero16
            return 0
        lax.fori_loop(0, HALF * (D // 16), z_flush, 0)

        seg0 = s * SEGS_PER_TILE
        pltpu.sync_copy(rows_v0.at[pl.ds(0, SEGS_PER_TILE)],
                        acc_sh.at[pl.ds(seg0, SEGS_PER_TILE)])
        plsc.subcore_barrier()

        start = c * CHUNKS_PER_SC + s * BASE_PER_TILE + jnp.minimum(s, EXTRA)
        count = BASE_PER_TILE + jnp.where(s < EXTRA, 1, 0)

        def issue(b, g):
            @pl.when(g < count)
            def _():
                gg = start + g
                pltpu.async_copy(ins_hbm.at[pl.ds(gg * CHUNK, CHUNK)],
                                 rows_bufs[b], rsems[b])
                pltpu.async_copy(ids_hbm.at[gg], idx_bufs[b], isems[b])

        issue(0, 0)
        issue(1, 1)

        def flush(rid, rcnt):
            @pl.when(rid >= 0)
            def _():
                for k in range(D // 16):
                    flush_v[0, pl.ds(k * 16, 16)] = run_v[pl.ds(k * 16, 16)]
                fidx_v[...] = jnp.full((HALF,), 0, jnp.int32) + rid
                pltpu.sync_copy(flush_v, acc_sh.at[fidx_v], add=True)
                w = cnt_v[pl.ds(rid, 16)]
                cnt_v[pl.ds(rid, 16)] = w + inc1 * rcnt.astype(jnp.float32)

        def fold(b, r0, nrows, a, st):
            rid, rcnt = st
            red = []
            for k in range(D // 16):
                v = rows_bufs[b][r0, pl.ds(k * 16, 16)]
                for r in range(1, nrows):
                    v = v + rows_bufs[b][r0 + r, pl.ds(k * 16, 16)]
                red.append(v)
            same = a == rid

            def same_fn():
                for k in range(D // 16):
                    run_v[pl.ds(k * 16, 16)] = (run_v[pl.ds(k * 16, 16)]
                                                + red[k])

            def diff_fn():
                flush(rid, rcnt)
                for k in range(D // 16):
                    run_v[pl.ds(k * 16, 16)] = red[k]

            lax.cond(same, same_fn, diff_fn)
            return (a, jnp.where(same, rcnt + nrows, nrows))

        def half_body(b, q, t16, span, st):
            a = span[0]
            z = span[15]
            r0 = q * SUB + t16 * HALF

            def uni16():
                return fold(b, r0, HALF, a, st)

            def mix16():
                flush(st[0], st[1])
                fidx_v[...] = span
                pltpu.sync_copy(rows_bufs[b].at[pl.ds(r0, HALF)],
                                acc_sh.at[fidx_v], add=True)
                for l in range(HALF):
                    v = span[l]
                    w = cnt_v[pl.ds(v, 16)]
                    cnt_v[pl.ds(v, 16)] = w + inc1
                return (jnp.int32(-1), jnp.int32(0))

            return lax.cond(a == z, uni16, mix16)

        def process_block(b, g, st_in):
            pltpu.make_async_copy(ins_hbm.at[pl.ds(0, CHUNK)],
                                  rows_bufs[b], rsems[b]).wait()
            pltpu.make_async_copy(ids_hbm.at[0],
                                  idx_bufs[b], isems[b]).wait()

            def q_body(q, st_q, _b=b):
                def span_body(t2, st, _q=q):
                    lo = idx_bufs[_b][_q, pl.ds(t2 * SPAN, 16)]
                    hi = idx_bufs[_b][_q, pl.ds(t2 * SPAN + HALF, 16)]
                    a = lo[0]
                    z = hi[15]

                    def uni32():
                        return fold(_b, _q * SUB + t2 * SPAN, SPAN, a, st)

                    def mix32():
                        st1 = half_body(_b, _q, t2 * 2, lo, st)
                        return half_body(_b, _q, t2 * 2 + 1, hi, st1)

                    return lax.cond(a == z, uni32, mix32)

                return lax.fori_loop(0, SUB // SPAN, span_body, st_q)

            st_out = lax.fori_loop(0, CHUNK // SUB, q_body, st_in)
            issue(b, g + 2)
            return st_out

        def outer(j2, st):
            for b in range(2):
                g = j2 * 2 + b
                st = lax.cond(
                    g < count,
                    functools.partial(process_block, b, g),
                    lambda x: x,
                    st)
            return st

        st0 = (jnp.int32(-1), jnp.int32(0))
        st = lax.fori_loop(0, (count + 1) // 2, outer, st0)
        flush(st[0], st[1])
        plsc.subcore_barrier()

        pltpu.sync_copy(acc_sh.at[pl.ds(seg0, SEGS_PER_TILE)],
                        rows_v0.at[pl.ds(0, SEGS_PER_TILE)])
        pltpu.sync_copy(rows_v0.at[pl.ds(0, SEGS_PER_TILE)],
                        sums_out.at[c, pl.ds(seg0, SEGS_PER_TILE)])
        pltpu.sync_copy(cnt_v.at[pl.ds(0, S)], cnt_out.at[c, s])

    return body


def _finalize_body(sp_ref, cp_ref, o_ref):
    sums = sp_ref[0] + sp_ref[1]
    cnts = jnp.sum(cp_ref[...], axis=(0, 1))
    o_ref[...] = sums / jnp.maximum(cnts[:, None], 1.0)


def kernel(ins, batch):
    ids = batch.astype(jnp.int32).reshape(NCHUNKS, CHUNK // SUB, SUB)
    sums_p, cnt_p = _accumulate()(ins, ids)
    return pl.pallas_call(
        _finalize_body,
        out_shape=jax.ShapeDtypeStruct((S, D), jnp.float32),
    )(sums_p, cnt_p)


# speedup vs baseline: 1.1643x; 1.1643x over previous
import functools

import jax
import jax.numpy as jnp
from jax import lax
from jax.experimental import pallas as pl
from jax.experimental.pallas import tpu as pltpu
from jax.experimental.pallas import tpu_sc as plsc

N = 320000
D = 128
S = 1024
CHUNK = 256
SUB = 128
SPAN = 32
HALF = 16
NCHUNKS = N // CHUNK
NC = 2
NS = 16
CHUNKS_PER_SC = NCHUNKS // NC
BASE_PER_TILE = CHUNKS_PER_SC // NS
EXTRA = CHUNKS_PER_SC - BASE_PER_TILE * NS
SEGS_PER_TILE = S // NS


def _accumulate():
    mesh = plsc.VectorSubcoreMesh(core_axis_name="c", subcore_axis_name="s")

    @functools.partial(
        pl.kernel,
        mesh=mesh,
        out_type=[
            jax.ShapeDtypeStruct((NC, S, D), jnp.float32),
            jax.ShapeDtypeStruct((NC, NS, S), jnp.float32),
        ],
        scratch_types=[
            pltpu.VMEM((CHUNK // SUB, SUB), jnp.int32),
            pltpu.VMEM((CHUNK // SUB, SUB), jnp.int32),
            pltpu.VMEM((CHUNK, D), jnp.float32),
            pltpu.VMEM((CHUNK, D), jnp.float32),
            pltpu.VMEM((S + 16,), jnp.float32),
            pltpu.VMEM((HALF, D), jnp.float32),
            pltpu.VMEM((HALF,), jnp.int32),
            pltpu.VMEM((D,), jnp.float32),
            pltpu.VMEM_SHARED((S, D), jnp.float32),
            pltpu.SemaphoreType.DMA,
            pltpu.SemaphoreType.DMA,
            pltpu.SemaphoreType.DMA,
            pltpu.SemaphoreType.DMA,
        ],
    )
    def body(ins_hbm, ids_hbm, sums_out, cnt_out,
             idx_v0, idx_v1, rows_v0, rows_v1, cnt_v, flush_v, fidx_v,
             run_v, acc_sh, rsem0, rsem1, isem0, isem1):
        c = lax.axis_index("c")
        s = lax.axis_index("s")
        zero16 = jnp.zeros((16,), jnp.float32)
        ilane = lax.iota(jnp.int32, 16)
        inc1 = jnp.where(ilane == 0, 1.0, 0.0).astype(jnp.float32)
        idx_bufs = (idx_v0, idx_v1)
        rows_bufs = (rows_v0, rows_v1)
        rsems = (rsem0, rsem1)
        isems = (isem0, isem1)

        def z_rows(i, _):
            rows_v0[i // 8, pl.ds((i % 8) * 16, 16)] = zero16
            return 0
        lax.fori_loop(0, SEGS_PER_TILE * (D // 16), z_rows, 0)

        def z_cnt(i, _):
            cnt_v[pl.ds(i * 16, 16)] = zero16
            return 0
        lax.fori_loop(0, (S + 16) // 16, z_cnt, 0)

        def z_flush(i, _):
            flush_v[i // 8, pl.ds((i % 8) * 16, 16)] = zero16
            return 0
        lax.fori_loop(0, HALF * (D // 16), z_flush, 0)

        seg0 = s * SEGS_PER_TILE
        pltpu.sync_copy(rows_v0.at[pl.ds(0, SEGS_PER_TILE)],
                        acc_sh.at[pl.ds(seg0, SEGS_PER_TILE)])
        plsc.subcore_barrier()

        start = c * CHUNKS_PER_SC + s * BASE_PER_TILE + jnp.minimum(s, EXTRA)
        count = BASE_PER_TILE + jnp.where(s < EXTRA, 1, 0)

        def issue(b, g):
            @pl.when(g < count)
            def _():
                gg = start + g
                pltpu.async_copy(ins_hbm.at[pl.ds(gg * CHUNK, CHUNK)],
                                 rows_bufs[b], rsems[b])
                pltpu.async_copy(ids_hbm.at[gg], idx_bufs[b], isems[b])

        issue(0, 0)
        issue(1, 1)

        def flush(rid, rcnt):
            @pl.when(rid >= 0)
            def _():
                for k in range(D // 16):
                    flush_v[0, pl.ds(k * 16, 16)] = run_v[pl.ds(k * 16, 16)]
                fidx_v[...] = jnp.full((HALF,), 0, jnp.int32) + rid
                pltpu.sync_copy(flush_v, acc_sh.at[fidx_v], add=True)
                w = cnt_v[pl.ds(rid, 16)]
                cnt_v[pl.ds(rid, 16)] = w + inc1 * rcnt.astype(jnp.float32)

        def fold(b, r0, nrows, a, st):
            rid, rcnt = st
            red = []
            for k in range(D // 16):
                v = rows_bufs[b][r0, pl.ds(k * 16, 16)]
                for r in range(1, nrows):
                    v = v + rows_bufs[b][r0 + r, pl.ds(k * 16, 16)]
                red.append(v)
            same = a == rid

            def same_fn():
                for k in range(D // 16):
                    run_v[pl.ds(k * 16, 16)] = (run_v[pl.ds(k * 16, 16)]
                                                + red[k])

            def diff_fn():
                flush(rid, rcnt)
                for k in range(D // 16):
                    run_v[pl.ds(k * 16, 16)] = red[k]

            lax.cond(same, same_fn, diff_fn)
            return (a, jnp.where(same, rcnt + nrows, nrows))

        def half_body(b, q, t16, span, st):
            a = span[0]
            z = span[15]
            r0 = q * SUB + t16 * HALF

            def uni16():
                return fold(b, r0, HALF, a, st)

            def mix16():
                flush(st[0], st[1])
                fidx_v[...] = span
                pltpu.sync_copy(rows_bufs[b].at[pl.ds(r0, HALF)],
                                acc_sh.at[fidx_v], add=True)
                for l in range(HALF):
                    v = span[l]
                    w = cnt_v[pl.ds(v, 16)]
                    cnt_v[pl.ds(v, 16)] = w + inc1
                return (jnp.int32(-1), jnp.int32(0))

            return lax.cond(a == z, uni16, mix16)

        def process_block(b, g, st_in):
            pltpu.make_async_copy(ins_hbm.at[pl.ds(0, CHUNK)],
                                  rows_bufs[b], rsems[b]).wait()
            pltpu.make_async_copy(ids_hbm.at[0],
                                  idx_bufs[b], isems[b]).wait()

            st_out = st_in
            for q in range(CHUNK // SUB):
                def span_body(t16, st, _q=q, _b=b):
                    span = idx_bufs[_b][_q, pl.ds(t16 * HALF, 16)]
                    return half_body(_b, _q, t16, span, st)

                st_out = lax.fori_loop(0, SUB // HALF, span_body, st_out)
            issue(b, g + 2)
            return st_out

        def outer(j2, st):
            for b in range(2):
                g = j2 * 2 + b
                st = lax.cond(
                    g < count,
                    functools.partial(process_block, b, g),
                    lambda x: x,
                    st)
            return st

        st0 = (jnp.int32(-1), jnp.int32(0))
        st = lax.fori_loop(0, (count + 1) // 2, outer, st0)
        flush(st[0], st[1])
        plsc.subcore_barrier()

        pltpu.sync_copy(acc_sh.at[pl.ds(seg0, SEGS_PER_TILE)],
                        rows_v0.at[pl.ds(0, SEGS_PER_TILE)])
        pltpu.sync_copy(rows_v0.at[pl.ds(0, SEGS_PER_TILE)],
                        sums_out.at[c, pl.ds(seg0, SEGS_PER_TILE)])
        pltpu.sync_copy(cnt_v.at[pl.ds(0, S)], cnt_out.at[c, s])

    return body


def _finalize_body(sp_ref, cp_ref, o_ref):
    sums = sp_ref[0] + sp_ref[1]
    cnts = jnp.sum(cp_ref[...], axis=(0, 1))
    o_ref[...] = sums / jnp.maximum(cnts[:, None], 1.0)


def kernel(ins, batch):
    ids = batch.astype(jnp.int32).reshape(NCHUNKS, CHUNK // SUB, SUB)
    sums_p, cnt_p = _accumulate()(ins, ids)
    return pl.pallas_call(
        _finalize_body,
        out_shape=jax.ShapeDtypeStruct((S, D), jnp.float32),
    )(sums_p, cnt_p)
